# Initial kernel scaffold; baseline (speedup 1.0000x reference)
import jax
import jax.numpy as jnp
from jax.experimental import pallas as pl

N_UAV = 512
N_TGT = 256
H = 64
HEADS = 4


def _row(v, h):
    return jax.lax.dot_general(v, h, (((1,), (1,)), ((), ())),
                               preferred_element_type=jnp.float32)


def _col(h, v):
    return jax.lax.dot_general(h, v, (((1,), (1,)), ((), ())),
                               preferred_element_type=jnp.float32)


def _att(hs, hd, a_s, a_d, mask_t):
    al = _row(a_s, hs)
    ar = _col(hd, a_d)
    e = al + ar
    e = jnp.where(e >= 0, e, 0.2 * e)
    e = jnp.where(mask_t, e, -jnp.inf)
    emax = jnp.max(e, axis=1, keepdims=True)
    emax = jnp.where(jnp.isfinite(emax), emax, 0.0)
    ee = jnp.exp(e - emax)
    den = jnp.sum(ee, axis=1, keepdims=True)
    out = jnp.dot(ee, hs, preferred_element_type=jnp.float32)
    return out / (den + 1e-16)


def _bn(x, g, b, n):
    mu = jnp.sum(x, axis=0, keepdims=True) / n
    var = jnp.sum((x - mu) ** 2, axis=0, keepdims=True) / n
    return (x - mu) * jax.lax.rsqrt(var + 1e-5) * g + b


def _elu(x):
    return jnp.where(x > 0, x, jnp.expm1(x))


def _fwd(uf_ref, tf_ref, uaT_ref, ta_ref, taT_ref,
         W1_ref, a1s_ref, a1d_ref, b1_ref, W2_ref, a2s_ref, a2d_ref, b2_ref,
         W3_ref, a3s_ref, a3d_ref, b3_ref, W4_ref, a4s_ref, a4d_ref, b4_ref,
         g1_ref, be1_ref, g2_ref, be2_ref, g3_ref, be3_ref, g4_ref, be4_ref,
         Wt_ref, bt_ref, Wf1_ref, bf1_ref, Wf2_ref, bf2_ref, out_ref):
    uf = uf_ref[:]
    tf = tf_ref[:]

    rows = jax.lax.broadcasted_iota(jnp.int32, (N_UAV, N_UAV), 0)
    cols = jax.lax.broadcasted_iota(jnp.int32, (N_UAV, N_UAV), 1)
    eye = rows == cols
    m1t = jnp.where(eye, True, uaT_ref[:] != 0.0)

    h1 = jnp.dot(uf, W1_ref[:], preferred_element_type=jnp.float32)
    a1s = a1s_ref[:]
    a1d = a1d_ref[:]
    outs = []
    for k in range(HEADS):
        hk = h1[:, k * H:(k + 1) * H]
        outs.append(_att(hk, hk, a1s[k:k + 1], a1d[k:k + 1], m1t))
    x1 = jnp.concatenate(outs, axis=1) + b1_ref[:]
    x1 = _elu(_bn(x1, g1_ref[:], be1_ref[:], float(N_UAV)))

    h2 = jnp.dot(x1, W2_ref[:], preferred_element_type=jnp.float32)
    x2 = _att(h2, h2, a2s_ref[:], a2d_ref[:], m1t) + b2_ref[:]
    uav_h = _elu(_bn(x2, g2_ref[:], be2_ref[:], float(N_UAV)))

    mt_u = ta_ref[:] != 0.0
    mt_t = taT_ref[:] != 0.0

    tproc = jnp.dot(tf, Wt_ref[:], preferred_element_type=jnp.float32) + bt_ref[:]
    h3u = jnp.dot(uf, W3_ref[:], preferred_element_type=jnp.float32)
    h3t = jnp.dot(tproc, W3_ref[:], preferred_element_type=jnp.float32)
    a3s = a3s_ref[:]
    a3d = a3d_ref[:]
    outs_u, outs_t = [], []
    for k in range(HEADS):
        hu = h3u[:, k * H:(k + 1) * H]
        ht = h3t[:, k * H:(k + 1) * H]
        outs_u.append(_att(ht, hu, a3s[k:k + 1], a3d[k:k + 1], mt_u))
        outs_t.append(_att(hu, ht, a3s[k:k + 1], a3d[k:k + 1], mt_t))
    y = jnp.concatenate(
        [jnp.concatenate(outs_u, axis=1), jnp.concatenate(outs_t, axis=1)],
        axis=0) + b3_ref[:]
    y = _elu(_bn(y, g3_ref[:], be3_ref[:], float(N_UAV + N_TGT)))

    h4 = jnp.dot(y, W4_ref[:], preferred_element_type=jnp.float32)
    h4u = h4[:N_UAV]
    h4t = h4[N_UAV:]
    y2u = _att(h4t, h4u, a4s_ref[:], a4d_ref[:], mt_u)
    y2t = _att(h4u, h4t, a4s_ref[:], a4d_ref[:], mt_t)
    y2 = jnp.concatenate([y2u, y2t], axis=0) + b4_ref[:]
    target_h = _elu(_bn(y2, g4_ref[:], be4_ref[:], float(N_UAV + N_TGT)))

    c = jnp.concatenate([uav_h, target_h[:N_UAV]], axis=1)
    hdn = jnp.dot(c, Wf1_ref[:], preferred_element_type=jnp.float32) + bf1_ref[:]
    hdn = jnp.maximum(hdn, 0.0)
    out_ref[:] = (jnp.dot(hdn, Wf2_ref[:], preferred_element_type=jnp.float32)
                  + bf2_ref[:])


def kernel(uav_features, target_features, uav_adj, target_adj, W1, a1s, a1d,
           b1, W2, a2s, a2d, b2, W3, a3s, a3d, b3, W4, a4s, a4d, b4, g1, be1,
           g2, be2, g3, be3, g4, be4, Wt, bt, Wf1, bf1, Wf2, bf2):
    row = lambda v: v.reshape(1, -1)
    args = (
        uav_features, target_features, uav_adj.T, target_adj, target_adj.T,
        W1, a1s, a1d, row(b1), W2, a2s, a2d, row(b2),
        W3, a3s, a3d, row(b3), W4, a4s, a4d, row(b4),
        row(g1), row(be1), row(g2), row(be2),
        row(g3), row(be3), row(g4), row(be4),
        Wt, row(bt), Wf1, row(bf1), Wf2, row(bf2),
    )
    return pl.pallas_call(
        _fwd,
        out_shape=jax.ShapeDtypeStruct((N_UAV, H // 2), jnp.float32),
    )(*args)


# speedup vs baseline: 2075.9531x; 2075.9531x over previous
import jax
import jax.numpy as jnp
from jax.experimental import pallas as pl

N_UAV = 512
N_TGT = 256
H = 64
HEADS = 4


def _row(v, h):
    return jax.lax.dot_general(v, h, (((1,), (1,)), ((), ())),
                               preferred_element_type=jnp.float32, precision=jax.lax.Precision.HIGHEST)


def _col(h, v):
    return jax.lax.dot_general(h, v, (((1,), (1,)), ((), ())),
                               preferred_element_type=jnp.float32, precision=jax.lax.Precision.HIGHEST)


def _att(hs, hd, a_s, a_d, mask_t):
    al = _row(a_s, hs)
    ar = _col(hd, a_d)
    e = al + ar
    e = jnp.where(e >= 0, e, 0.2 * e)
    e = jnp.where(mask_t, e, -jnp.inf)
    emax = jnp.max(e, axis=1, keepdims=True)
    emax = jnp.where(jnp.isfinite(emax), emax, 0.0)
    ee = jnp.exp(e - emax)
    den = jnp.sum(ee, axis=1, keepdims=True)
    out = jnp.dot(ee, hs, preferred_element_type=jnp.float32, precision=jax.lax.Precision.HIGHEST)
    return out / (den + 1e-16)


def _bn(x, g, b, n):
    mu = jnp.sum(x, axis=0, keepdims=True) / n
    var = jnp.sum((x - mu) ** 2, axis=0, keepdims=True) / n
    return (x - mu) * jax.lax.rsqrt(var + 1e-5) * g + b


def _elu(x):
    return jnp.where(x > 0, x, jnp.exp(x) - 1.0)


def _fwd(uf_ref, tf_ref, uaT_ref, ta_ref, taT_ref,
         W1_ref, a1s_ref, a1d_ref, b1_ref, W2_ref, a2s_ref, a2d_ref, b2_ref,
         W3_ref, a3s_ref, a3d_ref, b3_ref, W4_ref, a4s_ref, a4d_ref, b4_ref,
         g1_ref, be1_ref, g2_ref, be2_ref, g3_ref, be3_ref, g4_ref, be4_ref,
         Wt_ref, bt_ref, Wf1_ref, bf1_ref, Wf2_ref, bf2_ref, out_ref):
    uf = uf_ref[:]
    tf = tf_ref[:]

    rows = jax.lax.broadcasted_iota(jnp.int32, (N_UAV, N_UAV), 0)
    cols = jax.lax.broadcasted_iota(jnp.int32, (N_UAV, N_UAV), 1)
    eye = rows == cols
    m1t = jnp.logical_or(eye, uaT_ref[:] != 0.0)

    h1 = jnp.dot(uf, W1_ref[:], preferred_element_type=jnp.float32)
    a1s = a1s_ref[:]
    a1d = a1d_ref[:]
    outs = []
    for k in range(HEADS):
        hk = h1[:, k * H:(k + 1) * H]
        outs.append(_att(hk, hk, a1s[k:k + 1], a1d[k:k + 1], m1t))
    x1 = jnp.concatenate(outs, axis=1) + b1_ref[:]
    x1 = _elu(_bn(x1, g1_ref[:], be1_ref[:], float(N_UAV)))

    h2 = jnp.dot(x1, W2_ref[:], preferred_element_type=jnp.float32)
    x2 = _att(h2, h2, a2s_ref[:], a2d_ref[:], m1t) + b2_ref[:]
    uav_h = _elu(_bn(x2, g2_ref[:], be2_ref[:], float(N_UAV)))

    mt_u = ta_ref[:] != 0.0
    mt_t = taT_ref[:] != 0.0

    tproc = jnp.dot(tf, Wt_ref[:], preferred_element_type=jnp.float32) + bt_ref[:]
    h3u = jnp.dot(uf, W3_ref[:], preferred_element_type=jnp.float32)
    h3t = jnp.dot(tproc, W3_ref[:], preferred_element_type=jnp.float32)
    a3s = a3s_ref[:]
    a3d = a3d_ref[:]
    outs_u, outs_t = [], []
    for k in range(HEADS):
        hu = h3u[:, k * H:(k + 1) * H]
        ht = h3t[:, k * H:(k + 1) * H]
        outs_u.append(_att(ht, hu, a3s[k:k + 1], a3d[k:k + 1], mt_u))
        outs_t.append(_att(hu, ht, a3s[k:k + 1], a3d[k:k + 1], mt_t))
    y = jnp.concatenate(
        [jnp.concatenate(outs_u, axis=1), jnp.concatenate(outs_t, axis=1)],
        axis=0) + b3_ref[:]
    y = _elu(_bn(y, g3_ref[:], be3_ref[:], float(N_UAV + N_TGT)))

    h4 = jnp.dot(y, W4_ref[:], preferred_element_type=jnp.float32)
    h4u = h4[:N_UAV]
    h4t = h4[N_UAV:]
    y2u = _att(h4t, h4u, a4s_ref[:], a4d_ref[:], mt_u)
    y2t = _att(h4u, h4t, a4s_ref[:], a4d_ref[:], mt_t)
    y2 = jnp.concatenate([y2u, y2t], axis=0) + b4_ref[:]
    target_h = _elu(_bn(y2, g4_ref[:], be4_ref[:], float(N_UAV + N_TGT)))

    c = jnp.concatenate([uav_h, target_h[:N_UAV]], axis=1)
    hdn = jnp.dot(c, Wf1_ref[:], preferred_element_type=jnp.float32) + bf1_ref[:]
    hdn = jnp.maximum(hdn, 0.0)
    out_ref[:] = (jnp.dot(hdn, Wf2_ref[:], preferred_element_type=jnp.float32)
                  + bf2_ref[:])


def kernel(uav_features, target_features, uav_adj, target_adj, W1, a1s, a1d,
           b1, W2, a2s, a2d, b2, W3, a3s, a3d, b3, W4, a4s, a4d, b4, g1, be1,
           g2, be2, g3, be3, g4, be4, Wt, bt, Wf1, bf1, Wf2, bf2):
    row = lambda v: v.reshape(1, -1)
    args = (
        uav_features, target_features, uav_adj.T, target_adj, target_adj.T,
        W1, a1s, a1d, row(b1), W2, a2s, a2d, row(b2),
        W3, a3s, a3d, row(b3), W4, a4s, a4d, row(b4),
        row(g1), row(be1), row(g2), row(be2),
        row(g3), row(be3), row(g4), row(be4),
        Wt, row(bt), Wf1, row(bf1), Wf2, row(bf2),
    )
    return pl.pallas_call(
        _fwd,
        out_shape=jax.ShapeDtypeStruct((N_UAV, H // 2), jnp.float32),
    )(*args)


# speedup vs baseline: 2503.0601x; 1.2057x over previous
import jax
import jax.numpy as jnp
from jax.experimental import pallas as pl

N_UAV = 512
N_TGT = 256
H = 64
HEADS = 4

_F32 = jnp.float32
_HI = jax.lax.Precision.HIGHEST


def _row(v, h):
    return jax.lax.dot_general(v, h, (((1,), (1,)), ((), ())),
                               preferred_element_type=_F32, precision=_HI)


def _col(h, v):
    return jax.lax.dot_general(h, v, (((1,), (1,)), ((), ())),
                               preferred_element_type=_F32, precision=_HI)


def _att_dm(hs, hd, a_s, a_d, mask):
    al = _row(a_s, hs)
    ar = _col(hd, a_d)
    e = al + ar
    e = jnp.where(e >= 0, e, 0.2 * e)
    e = jnp.where(mask, e, -jnp.inf)
    emax = jnp.max(e, axis=1, keepdims=True)
    emax = jnp.where(jnp.isfinite(emax), emax, 0.0)
    ee = jnp.exp(e - emax)
    den = jnp.sum(ee, axis=1, keepdims=True)
    out = jnp.dot(ee, hs, preferred_element_type=_F32, precision=_HI)
    return out / (den + 1e-16)


def _att_sm(hs, hd, a_s, a_d, mask):
    al = _col(hs, a_s)
    ar = _row(a_d, hd)
    e = al + ar
    e = jnp.where(e >= 0, e, 0.2 * e)
    e = jnp.where(mask, e, -jnp.inf)
    emax = jnp.max(e, axis=0, keepdims=True)
    emax = jnp.where(jnp.isfinite(emax), emax, 0.0)
    ee = jnp.exp(e - emax)
    den = jnp.sum(ee, axis=0, keepdims=True)
    alpha = ee / (den + 1e-16)
    return jax.lax.dot_general(alpha, hs, (((0,), (0,)), ((), ())),
                               preferred_element_type=_F32, precision=_HI)


def _bn(x, g, b, n):
    mu = jnp.sum(x, axis=0, keepdims=True) / n
    var = jnp.sum((x - mu) ** 2, axis=0, keepdims=True) / n
    return (x - mu) * jax.lax.rsqrt(var + 1e-5) * g + b


def _elu(x):
    return jnp.where(x > 0, x, jnp.exp(x) - 1.0)


def _fwd(uf_ref, tf_ref, ua_ref, ta_ref,
         W1_ref, a1s_ref, a1d_ref, b1_ref, W2_ref, a2s_ref, a2d_ref, b2_ref,
         W3_ref, a3s_ref, a3d_ref, b3_ref, W4_ref, a4s_ref, a4d_ref, b4_ref,
         g1_ref, be1_ref, g2_ref, be2_ref, g3_ref, be3_ref, g4_ref, be4_ref,
         Wt_ref, bt_ref, Wf1_ref, bf1_ref, Wf2_ref, bf2_ref, out_ref):
    uf = uf_ref[:]
    tf = tf_ref[:]

    rows = jax.lax.broadcasted_iota(jnp.int32, (N_UAV, N_UAV), 0)
    cols = jax.lax.broadcasted_iota(jnp.int32, (N_UAV, N_UAV), 1)
    m1 = jnp.logical_or(rows == cols, ua_ref[:] != 0.0)

    h1 = jnp.dot(uf, W1_ref[:], preferred_element_type=_F32)
    a1s = a1s_ref[:]
    a1d = a1d_ref[:]
    outs = []
    for k in range(HEADS):
        hk = h1[:, k * H:(k + 1) * H]
        outs.append(_att_sm(hk, hk, a1s[k:k + 1], a1d[k:k + 1], m1))
    x1 = jnp.concatenate(outs, axis=1) + b1_ref[:]
    x1 = _elu(_bn(x1, g1_ref[:], be1_ref[:], float(N_UAV)))

    h2 = jnp.dot(x1, W2_ref[:], preferred_element_type=_F32)
    x2 = _att_sm(h2, h2, a2s_ref[:], a2d_ref[:], m1) + b2_ref[:]
    uav_h = _elu(_bn(x2, g2_ref[:], be2_ref[:], float(N_UAV)))

    mt = ta_ref[:] != 0.0

    tproc = jnp.dot(tf, Wt_ref[:], preferred_element_type=_F32) + bt_ref[:]
    h3u = jnp.dot(uf, W3_ref[:], preferred_element_type=_F32)
    h3t = jnp.dot(tproc, W3_ref[:], preferred_element_type=_F32)
    a3s = a3s_ref[:]
    a3d = a3d_ref[:]
    outs_u, outs_t = [], []
    for k in range(HEADS):
        hu = h3u[:, k * H:(k + 1) * H]
        ht = h3t[:, k * H:(k + 1) * H]
        outs_u.append(_att_dm(ht, hu, a3s[k:k + 1], a3d[k:k + 1], mt))
        outs_t.append(_att_sm(hu, ht, a3s[k:k + 1], a3d[k:k + 1], mt))
    y = jnp.concatenate(
        [jnp.concatenate(outs_u, axis=1), jnp.concatenate(outs_t, axis=1)],
        axis=0) + b3_ref[:]
    y = _elu(_bn(y, g3_ref[:], be3_ref[:], float(N_UAV + N_TGT)))

    h4 = jnp.dot(y, W4_ref[:], preferred_element_type=_F32)
    h4u = h4[:N_UAV]
    h4t = h4[N_UAV:]
    y2u = _att_dm(h4t, h4u, a4s_ref[:], a4d_ref[:], mt)
    y2t = _att_sm(h4u, h4t, a4s_ref[:], a4d_ref[:], mt)
    y2 = jnp.concatenate([y2u, y2t], axis=0) + b4_ref[:]
    target_h = _elu(_bn(y2, g4_ref[:], be4_ref[:], float(N_UAV + N_TGT)))

    c = jnp.concatenate([uav_h, target_h[:N_UAV]], axis=1)
    hdn = jnp.dot(c, Wf1_ref[:], preferred_element_type=_F32) + bf1_ref[:]
    hdn = jnp.maximum(hdn, 0.0)
    out_ref[:] = (jnp.dot(hdn, Wf2_ref[:], preferred_element_type=_F32)
                  + bf2_ref[:])


def kernel(uav_features, target_features, uav_adj, target_adj, W1, a1s, a1d,
           b1, W2, a2s, a2d, b2, W3, a3s, a3d, b3, W4, a4s, a4d, b4, g1, be1,
           g2, be2, g3, be3, g4, be4, Wt, bt, Wf1, bf1, Wf2, bf2):
    row = lambda v: v.reshape(1, -1)
    args = (
        uav_features, target_features, uav_adj, target_adj,
        W1, a1s, a1d, row(b1), W2, a2s, a2d, row(b2),
        W3, a3s, a3d, row(b3), W4, a4s, a4d, row(b4),
        row(g1), row(be1), row(g2), row(be2),
        row(g3), row(be3), row(g4), row(be4),
        Wt, row(bt), Wf1, row(bf1), Wf2, row(bf2),
    )
    return pl.pallas_call(
        _fwd,
        out_shape=jax.ShapeDtypeStruct((N_UAV, H // 2), jnp.float32),
    )(*args)


# speedup vs baseline: 2942.9984x; 1.1758x over previous
import jax
import jax.numpy as jnp
from jax.experimental import pallas as pl

N_UAV = 512
N_TGT = 256
H = 64
HEADS = 4

_F32 = jnp.float32
_HI = jax.lax.Precision.HIGHEST


def _row(v, h):
    return jax.lax.dot_general(v, h, (((1,), (1,)), ((), ())),
                               preferred_element_type=_F32, precision=_HI)


def _col(h, v):
    return jax.lax.dot_general(h, v, (((1,), (1,)), ((), ())),
                               preferred_element_type=_F32, precision=_HI)


def _dot3(a, b, dn):
    ah = a.astype(jnp.bfloat16)
    al = (a - ah.astype(_F32)).astype(jnp.bfloat16)
    bh = b.astype(jnp.bfloat16)
    bl = (b - bh.astype(_F32)).astype(jnp.bfloat16)
    d = lambda x, y: jax.lax.dot_general(x, y, dn, preferred_element_type=_F32)
    return d(ah, bh) + d(al, bh) + d(ah, bl)


_DN_NN = (((1,), (0,)), ((), ()))
_DN_TN = (((0,), (0,)), ((), ()))


def _att_dm(hs, hd, a_s, a_d, maskf):
    al = _row(a_s, hs)
    ar = _col(hd, a_d)
    e = al + ar
    e = jnp.maximum(e, 0.2 * e)
    emax = jnp.max(e, axis=1, keepdims=True)
    ee = jnp.exp(e - emax) * maskf
    den = jnp.sum(ee, axis=1, keepdims=True)
    out = _dot3(ee, hs, _DN_NN)
    return out / (den + 1e-16)


def _att_sm(hs, hd, a_s, a_d, maskf):
    al = _col(hs, a_s)
    ar = _row(a_d, hd)
    e = al + ar
    e = jnp.maximum(e, 0.2 * e)
    emax = jnp.max(e, axis=0, keepdims=True)
    ee = jnp.exp(e - emax) * maskf
    den = jnp.sum(ee, axis=0, keepdims=True)
    alpha = ee / (den + 1e-16)
    return _dot3(alpha, hs, _DN_TN)


def _bn(x, g, b, n):
    mu = jnp.sum(x, axis=0, keepdims=True) / n
    var = jnp.sum((x - mu) ** 2, axis=0, keepdims=True) / n
    return (x - mu) * jax.lax.rsqrt(var + 1e-5) * g + b


def _elu(x):
    return jnp.where(x > 0, x, jnp.exp(x) - 1.0)


def _fwd(uf_ref, tf_ref, ua_ref, ta_ref,
         W1_ref, a1s_ref, a1d_ref, b1_ref, W2_ref, a2s_ref, a2d_ref, b2_ref,
         W3_ref, a3s_ref, a3d_ref, b3_ref, W4_ref, a4s_ref, a4d_ref, b4_ref,
         g1_ref, be1_ref, g2_ref, be2_ref, g3_ref, be3_ref, g4_ref, be4_ref,
         Wt_ref, bt_ref, Wf1_ref, bf1_ref, Wf2_ref, bf2_ref, out_ref):
    uf = uf_ref[:]
    tf = tf_ref[:]

    rows = jax.lax.broadcasted_iota(jnp.int32, (N_UAV, N_UAV), 0)
    cols = jax.lax.broadcasted_iota(jnp.int32, (N_UAV, N_UAV), 1)
    m1 = jnp.where(jnp.logical_or(rows == cols, ua_ref[:] != 0.0), 1.0, 0.0)

    h1 = jnp.dot(uf, W1_ref[:], preferred_element_type=_F32)
    a1s = a1s_ref[:]
    a1d = a1d_ref[:]
    outs = []
    for k in range(HEADS):
        hk = h1[:, k * H:(k + 1) * H]
        outs.append(_att_sm(hk, hk, a1s[k:k + 1], a1d[k:k + 1], m1))
    x1 = jnp.concatenate(outs, axis=1) + b1_ref[:]
    x1 = _elu(_bn(x1, g1_ref[:], be1_ref[:], float(N_UAV)))

    h2 = jnp.dot(x1, W2_ref[:], preferred_element_type=_F32)
    x2 = _att_sm(h2, h2, a2s_ref[:], a2d_ref[:], m1) + b2_ref[:]
    uav_h = _elu(_bn(x2, g2_ref[:], be2_ref[:], float(N_UAV)))

    mt = jnp.where(ta_ref[:] != 0.0, 1.0, 0.0)

    tproc = jnp.dot(tf, Wt_ref[:], preferred_element_type=_F32) + bt_ref[:]
    h3u = jnp.dot(uf, W3_ref[:], preferred_element_type=_F32)
    h3t = jnp.dot(tproc, W3_ref[:], preferred_element_type=_F32)
    a3s = a3s_ref[:]
    a3d = a3d_ref[:]
    outs_u, outs_t = [], []
    for k in range(HEADS):
        hu = h3u[:, k * H:(k + 1) * H]
        ht = h3t[:, k * H:(k + 1) * H]
        outs_u.append(_att_dm(ht, hu, a3s[k:k + 1], a3d[k:k + 1], mt))
        outs_t.append(_att_sm(hu, ht, a3s[k:k + 1], a3d[k:k + 1], mt))
    y = jnp.concatenate(
        [jnp.concatenate(outs_u, axis=1), jnp.concatenate(outs_t, axis=1)],
        axis=0) + b3_ref[:]
    y = _elu(_bn(y, g3_ref[:], be3_ref[:], float(N_UAV + N_TGT)))

    h4 = jnp.dot(y, W4_ref[:], preferred_element_type=_F32)
    h4u = h4[:N_UAV]
    h4t = h4[N_UAV:]
    y2u = _att_dm(h4t, h4u, a4s_ref[:], a4d_ref[:], mt)
    y2t = _att_sm(h4u, h4t, a4s_ref[:], a4d_ref[:], mt)
    y2 = jnp.concatenate([y2u, y2t], axis=0) + b4_ref[:]
    target_h = _elu(_bn(y2, g4_ref[:], be4_ref[:], float(N_UAV + N_TGT)))

    c = jnp.concatenate([uav_h, target_h[:N_UAV]], axis=1)
    hdn = jnp.dot(c, Wf1_ref[:], preferred_element_type=_F32) + bf1_ref[:]
    hdn = jnp.maximum(hdn, 0.0)
    out_ref[:] = (jnp.dot(hdn, Wf2_ref[:], preferred_element_type=_F32)
                  + bf2_ref[:])


def kernel(uav_features, target_features, uav_adj, target_adj, W1, a1s, a1d,
           b1, W2, a2s, a2d, b2, W3, a3s, a3d, b3, W4, a4s, a4d, b4, g1, be1,
           g2, be2, g3, be3, g4, be4, Wt, bt, Wf1, bf1, Wf2, bf2):
    row = lambda v: v.reshape(1, -1)
    args = (
        uav_features, target_features, uav_adj, target_adj,
        W1, a1s, a1d, row(b1), W2, a2s, a2d, row(b2),
        W3, a3s, a3d, row(b3), W4, a4s, a4d, row(b4),
        row(g1), row(be1), row(g2), row(be2),
        row(g3), row(be3), row(g4), row(be4),
        Wt, row(bt), Wf1, row(bf1), Wf2, row(bf2),
    )
    return pl.pallas_call(
        _fwd,
        out_shape=jax.ShapeDtypeStruct((N_UAV, H // 2), jnp.float32),
    )(*args)
